# baseline (speedup 1.0000x reference)
import functools

import numpy as np
import jax
import jax.numpy as jnp
from jax.experimental import pallas as pl
from jax.experimental.pallas import tpu as pltpu

_EPS = 1e-5


def _bn_fold(gamma, beta, mean, var, eps=_EPS):
    scale = gamma / jnp.sqrt(var + eps)
    shift = beta - mean * scale
    return scale.astype(jnp.float32), shift.astype(jnp.float32)


def _tap_masks_hw(H, W):
    hh = np.arange(H)[:, None]
    ww = np.arange(W)[None, :]
    rows = []
    for kh in range(3):
        for kw in range(3):
            dh, dw = kh - 1, kw - 1
            m = ((hh + dh >= 0) & (hh + dh < H) &
                 (ww + dw >= 0) & (ww + dw < W))
            rows.append(m.reshape(-1))
    return np.stack(rows, axis=0).astype(np.float32)


def _linker_kernel(W, Cmid, x_ref, w1_ref, b1_ref, mask_ref, w3_ref,
                   b3_ref, o_ref, t_ref):
    HW = x_ref.shape[2]
    y = jnp.maximum(
        jnp.dot(w1_ref[...], x_ref[0],
                preferred_element_type=jnp.float32) + b1_ref[...], 0.0)
    yb = y.astype(jnp.bfloat16)
    mask = mask_ref[...]
    for kh in range(3):
        for kw in range(3):
            t = kh * 3 + kw
            off = (kh - 1) * W + (kw - 1)
            if off == 0:
                shifted = yb
            else:
                shifted = pltpu.roll(yb, shift=(-off) % HW, axis=1)
                shifted = shifted * mask[t:t + 1, :]
            t_ref[t * Cmid:(t + 1) * Cmid, :] = shifted
    acc = jnp.dot(w3_ref[...], t_ref[...],
                  preferred_element_type=jnp.float32) + b3_ref[...]
    o_ref[0] = jnp.maximum(acc, 0.0)


def _linker(x_nchw, w1, bn2d, w3, bn3d):
    N, Cin, H, Wd = x_nchw.shape
    Cmid = w1.shape[1]
    D = H
    HW = H * Wd

    s2, b2 = _bn_fold(*bn2d)
    s3, b3 = _bn_fold(*bn3d)

    w1_eff = (jnp.transpose(w1) * s2[:, None]).astype(jnp.float32)
    b1c = b2.reshape(Cmid, 1)

    w3f = w3.astype(jnp.float32)
    variants = (w3f[1] + w3f[2], w3f[0] + w3f[1] + w3f[2], w3f[0] + w3f[1])

    def to_mat(vw):
        return (jnp.transpose(vw, (3, 0, 1, 2)).reshape(Cmid, 9 * Cmid)
                * s3[:, None])

    w3_all = (jnp.stack([to_mat(v) for v in variants], axis=1)
              .reshape(3 * Cmid, 9 * Cmid).astype(jnp.bfloat16))
    b3_all = jnp.repeat(b3, 3).reshape(3 * Cmid, 1)

    x3 = x_nchw.reshape(N, Cin, HW).astype(jnp.float32)
    masks = jnp.asarray(_tap_masks_hw(H, Wd)).astype(jnp.bfloat16)

    out = pl.pallas_call(
        functools.partial(_linker_kernel, Wd, Cmid),
        out_shape=jax.ShapeDtypeStruct((N, 3 * Cmid, HW), jnp.float32),
        grid=(N,),
        in_specs=[
            pl.BlockSpec((1, Cin, HW), lambda n: (n, 0, 0)),
            pl.BlockSpec((Cmid, Cin), lambda n: (0, 0)),
            pl.BlockSpec((Cmid, 1), lambda n: (0, 0)),
            pl.BlockSpec((9, HW), lambda n: (0, 0)),
            pl.BlockSpec((3 * Cmid, 9 * Cmid), lambda n: (0, 0)),
            pl.BlockSpec((3 * Cmid, 1), lambda n: (0, 0)),
        ],
        out_specs=pl.BlockSpec((1, 3 * Cmid, HW), lambda n: (n, 0, 0)),
        scratch_shapes=[pltpu.VMEM((9 * Cmid, HW), jnp.bfloat16)],
        compiler_params=pltpu.CompilerParams(
            dimension_semantics=("parallel",)),
    )(x3, w1_eff, b1c, masks, w3_all, b3_all)

    o5 = out.reshape(N, Cmid, 3, H, Wd)
    full = jnp.concatenate([
        o5[:, :, 0:1],
        jnp.broadcast_to(o5[:, :, 1:2], (N, Cmid, D - 2, H, Wd)),
        o5[:, :, 2:3],
    ], axis=2)
    return full


def _base_kernel(x_ref, w_ref, b_ref, o_ref):
    acc = jnp.dot(x_ref[...], w_ref[...], preferred_element_type=jnp.float32)
    o_ref[...] = jnp.maximum(acc + b_ref[...], 0.0)


def _base_matmul(x, w, b, tn=2048):
    M, K = x.shape
    K2, Nc = w.shape
    assert K == K2
    tn = min(tn, Nc)
    if Nc % tn != 0:
        tn = Nc
    b2 = jnp.reshape(b, (1, Nc)).astype(jnp.float32)
    return pl.pallas_call(
        _base_kernel,
        out_shape=jax.ShapeDtypeStruct((M, Nc), jnp.float32),
        grid=(Nc // tn,),
        in_specs=[
            pl.BlockSpec((M, K), lambda j: (0, 0)),
            pl.BlockSpec((K, tn), lambda j: (0, j)),
            pl.BlockSpec((1, tn), lambda j: (0, j)),
        ],
        out_specs=pl.BlockSpec((M, tn), lambda j: (0, j)),
        compiler_params=pltpu.CompilerParams(
            dimension_semantics=("parallel",)),
    )(x.astype(jnp.float32), w.astype(jnp.float32), b2)


def kernel(l0_w1, l0_bn2d_gamma, l0_bn2d_beta, l0_bn2d_mean, l0_bn2d_var,
           l0_w3, l0_bn3d_gamma, l0_bn3d_beta, l0_bn3d_mean, l0_bn3d_var,
           l1_w1, l1_bn2d_gamma, l1_bn2d_beta, l1_bn2d_mean, l1_bn2d_var,
           l1_w3, l1_bn3d_gamma, l1_bn3d_beta, l1_bn3d_mean, l1_bn3d_var,
           base_w, base_b, feat0, feat1, final_vector):
    out0 = _linker(feat0, l0_w1,
                   (l0_bn2d_gamma, l0_bn2d_beta, l0_bn2d_mean, l0_bn2d_var),
                   l0_w3,
                   (l0_bn3d_gamma, l0_bn3d_beta, l0_bn3d_mean, l0_bn3d_var))
    out1 = _linker(feat1, l1_w1,
                   (l1_bn2d_gamma, l1_bn2d_beta, l1_bn2d_mean, l1_bn2d_var),
                   l1_w3,
                   (l1_bn3d_gamma, l1_bn3d_beta, l1_bn3d_mean, l1_bn3d_var))
    N = final_vector.shape[0]
    flat = final_vector.reshape(N, -1)
    x = _base_matmul(flat, base_w, base_b)
    return x, [out0, out1]


# speedup vs baseline: 1.3449x; 1.3449x over previous
import functools

import numpy as np
import jax
import jax.numpy as jnp
from jax.experimental import pallas as pl
from jax.experimental.pallas import tpu as pltpu

_EPS = 1e-5


def _bn_fold(gamma, beta, mean, var, eps=_EPS):
    scale = gamma / jnp.sqrt(var + eps)
    shift = beta - mean * scale
    return scale.astype(jnp.float32), shift.astype(jnp.float32)


def _tap_masks_hw(H, W):
    hh = np.arange(H)[:, None]
    ww = np.arange(W)[None, :]
    rows = []
    for kh in range(3):
        for kw in range(3):
            dh, dw = kh - 1, kw - 1
            m = ((hh + dh >= 0) & (hh + dh < H) &
                 (ww + dw >= 0) & (ww + dw < W))
            rows.append(m.reshape(-1))
    return np.stack(rows, axis=0).astype(np.float32)


def _linker_kernel(W, Cmid, x_ref, w1_ref, b1_ref, mask_ref, w3_ref,
                   b3_ref, o_ref, t_ref):
    HW = x_ref.shape[2]
    y = jnp.maximum(
        jnp.dot(w1_ref[...], x_ref[0],
                preferred_element_type=jnp.float32) + b1_ref[...], 0.0)
    yb = y.astype(jnp.bfloat16)
    mask = mask_ref[...]
    for kh in range(3):
        for kw in range(3):
            t = kh * 3 + kw
            off = (kh - 1) * W + (kw - 1)
            if off == 0:
                shifted = yb
            else:
                shifted = pltpu.roll(yb, shift=(-off) % HW, axis=1)
                shifted = shifted * mask[t:t + 1, :]
            t_ref[t * Cmid:(t + 1) * Cmid, :] = shifted
    acc = jnp.dot(w3_ref[...], t_ref[...],
                  preferred_element_type=jnp.float32) + b3_ref[...]
    o_ref[...] = jnp.maximum(acc, 0.0)


def _linker(x_nchw, w1, bn2d, w3, bn3d):
    N, Cin, H, Wd = x_nchw.shape
    Cmid = w1.shape[1]
    D = H
    HW = H * Wd

    s2, b2 = _bn_fold(*bn2d)
    s3, b3 = _bn_fold(*bn3d)

    w1_eff = (jnp.transpose(w1) * s2[:, None]).astype(jnp.float32)
    b1c = b2.reshape(Cmid, 1)

    w3f = w3.astype(jnp.float32)
    variants = (w3f[1] + w3f[2], w3f[0] + w3f[1] + w3f[2], w3f[0] + w3f[1])

    def to_mat(vw):
        return (jnp.transpose(vw, (3, 0, 1, 2)).reshape(Cmid, 9 * Cmid)
                * s3[:, None])

    w3_all = jnp.concatenate([to_mat(v) for v in variants],
                             axis=0).astype(jnp.bfloat16)
    b3_all = jnp.tile(b3.reshape(Cmid, 1), (3, 1))

    x3 = x_nchw.reshape(N, Cin, HW).astype(jnp.float32)
    masks = jnp.asarray(_tap_masks_hw(H, Wd)).astype(jnp.bfloat16)

    out = pl.pallas_call(
        functools.partial(_linker_kernel, Wd, Cmid),
        out_shape=jax.ShapeDtypeStruct((3 * Cmid, N * HW), jnp.float32),
        grid=(N,),
        in_specs=[
            pl.BlockSpec((1, Cin, HW), lambda n: (n, 0, 0)),
            pl.BlockSpec((Cmid, Cin), lambda n: (0, 0)),
            pl.BlockSpec((Cmid, 1), lambda n: (0, 0)),
            pl.BlockSpec((9, HW), lambda n: (0, 0)),
            pl.BlockSpec((3 * Cmid, 9 * Cmid), lambda n: (0, 0)),
            pl.BlockSpec((3 * Cmid, 1), lambda n: (0, 0)),
        ],
        out_specs=pl.BlockSpec((3 * Cmid, HW), lambda n: (0, n)),
        scratch_shapes=[pltpu.VMEM((9 * Cmid, HW), jnp.bfloat16)],
        compiler_params=pltpu.CompilerParams(
            dimension_semantics=("parallel",)),
    )(x3, w1_eff, b1c, masks, w3_all, b3_all)

    out5 = out.reshape(3, Cmid, N, H, Wd)
    vidx = np.ones((D,), np.int32)
    vidx[0] = 0
    vidx[-1] = 2
    full = jnp.take(out5, jnp.asarray(vidx), axis=0)
    return jnp.transpose(full, (2, 1, 0, 3, 4))


def _base_kernel(x_ref, w_ref, b_ref, o_ref):
    acc = jnp.dot(x_ref[...], w_ref[...], preferred_element_type=jnp.float32)
    o_ref[...] = jnp.maximum(acc + b_ref[...], 0.0)


def _base_matmul(x, w, b, tn=2048):
    M, K = x.shape
    K2, Nc = w.shape
    assert K == K2
    tn = min(tn, Nc)
    if Nc % tn != 0:
        tn = Nc
    b2 = jnp.reshape(b, (1, Nc)).astype(jnp.float32)
    return pl.pallas_call(
        _base_kernel,
        out_shape=jax.ShapeDtypeStruct((M, Nc), jnp.float32),
        grid=(Nc // tn,),
        in_specs=[
            pl.BlockSpec((M, K), lambda j: (0, 0)),
            pl.BlockSpec((K, tn), lambda j: (0, j)),
            pl.BlockSpec((1, tn), lambda j: (0, j)),
        ],
        out_specs=pl.BlockSpec((M, tn), lambda j: (0, j)),
        compiler_params=pltpu.CompilerParams(
            dimension_semantics=("parallel",)),
    )(x.astype(jnp.float32), w.astype(jnp.float32), b2)


def kernel(l0_w1, l0_bn2d_gamma, l0_bn2d_beta, l0_bn2d_mean, l0_bn2d_var,
           l0_w3, l0_bn3d_gamma, l0_bn3d_beta, l0_bn3d_mean, l0_bn3d_var,
           l1_w1, l1_bn2d_gamma, l1_bn2d_beta, l1_bn2d_mean, l1_bn2d_var,
           l1_w3, l1_bn3d_gamma, l1_bn3d_beta, l1_bn3d_mean, l1_bn3d_var,
           base_w, base_b, feat0, feat1, final_vector):
    out0 = _linker(feat0, l0_w1,
                   (l0_bn2d_gamma, l0_bn2d_beta, l0_bn2d_mean, l0_bn2d_var),
                   l0_w3,
                   (l0_bn3d_gamma, l0_bn3d_beta, l0_bn3d_mean, l0_bn3d_var))
    out1 = _linker(feat1, l1_w1,
                   (l1_bn2d_gamma, l1_bn2d_beta, l1_bn2d_mean, l1_bn2d_var),
                   l1_w3,
                   (l1_bn3d_gamma, l1_bn3d_beta, l1_bn3d_mean, l1_bn3d_var))
    N = final_vector.shape[0]
    flat = final_vector.reshape(N, -1)
    x = _base_matmul(flat, base_w, base_b)
    return x, [out0, out1]


# speedup vs baseline: 3.8925x; 2.8943x over previous
import functools

import numpy as np
import jax
import jax.numpy as jnp
from jax.experimental import pallas as pl
from jax.experimental.pallas import tpu as pltpu

_EPS = 1e-5


def _tap_masks_hw(H, W):
    hh = np.arange(H)[:, None]
    ww = np.arange(W)[None, :]
    cols = []
    for kh in range(3):
        for kw in range(3):
            dh, dw = kh - 1, kw - 1
            m = ((hh + dh >= 0) & (hh + dh < H) &
                 (ww + dw >= 0) & (ww + dw < W))
            cols.append(m.reshape(-1))
    return np.stack(cols, axis=1).astype(np.float32)


def _linker_kernel(W, D, Cmid, Dt, x_ref, w1_ref, bn2_ref, mask_ref, w3_ref,
                   bn3_ref, o_ref, t_ref, o3_ref):
    j = pl.program_id(1)
    HW = x_ref.shape[1]

    @pl.when(j == 0)
    def _compute():
        s2 = bn2_ref[0:1] * jax.lax.rsqrt(bn2_ref[3:4] + _EPS)
        b2 = bn2_ref[1:2] - bn2_ref[2:3] * s2
        y = jnp.maximum(
            jnp.dot(x_ref[0], w1_ref[...],
                    preferred_element_type=jnp.float32) * s2 + b2, 0.0)
        yb = y.astype(jnp.bfloat16)
        for kh in range(3):
            for kw in range(3):
                t = kh * 3 + kw
                off = (kh - 1) * W + (kw - 1)
                if off == 0:
                    shifted = yb
                else:
                    shifted = pltpu.roll(yb, shift=(-off) % HW, axis=0)
                    shifted = shifted * mask_ref[:, t:t + 1]
                t_ref[:, t * Cmid:(t + 1) * Cmid] = shifted
        tv = t_ref[...]
        p0 = jnp.dot(tv, w3_ref[0], preferred_element_type=jnp.float32)
        p1 = jnp.dot(tv, w3_ref[1], preferred_element_type=jnp.float32)
        p2 = jnp.dot(tv, w3_ref[2], preferred_element_type=jnp.float32)
        sv = bn3_ref[0:1] * jax.lax.rsqrt(bn3_ref[3:4] + _EPS)
        bv = bn3_ref[1:2] - bn3_ref[2:3] * sv
        o3_ref[:, 0:Cmid] = jnp.maximum((p1 + p2) * sv + bv, 0.0)
        o3_ref[:, Cmid:2 * Cmid] = jnp.maximum(
            (p0 + p1 + p2) * sv + bv, 0.0)
        o3_ref[:, 2 * Cmid:3 * Cmid] = jnp.maximum((p0 + p1) * sv + bv, 0.0)

    o_ref[0] = jnp.broadcast_to(o3_ref[:, Cmid:2 * Cmid][None],
                                (Dt, HW, Cmid))

    @pl.when(j == 0)
    def _front():
        o_ref[0, 0] = o3_ref[:, 0:Cmid]

    @pl.when(j == D // Dt - 1)
    def _back():
        o_ref[0, Dt - 1] = o3_ref[:, 2 * Cmid:3 * Cmid]


def _linker(x_nchw, w1, bn2d, w3, bn3d):
    N, Cin, H, Wd = x_nchw.shape
    Cmid = w1.shape[1]
    D = H
    HW = H * Wd

    bn2 = jnp.stack(bn2d, axis=0)
    bn3 = jnp.stack(bn3d, axis=0)

    w3r = w3.reshape(3, 9 * Cmid, Cmid).astype(jnp.bfloat16)

    x3 = x_nchw.transpose(0, 2, 3, 1).reshape(N, HW, Cin).astype(jnp.float32)
    masks = jnp.asarray(_tap_masks_hw(H, Wd)).astype(jnp.bfloat16)

    Dt = 16
    while D % Dt:
        Dt //= 2

    out = pl.pallas_call(
        functools.partial(_linker_kernel, Wd, D, Cmid, Dt),
        out_shape=jax.ShapeDtypeStruct((N, D, HW, Cmid), jnp.float32),
        grid=(N, D // Dt),
        in_specs=[
            pl.BlockSpec((1, HW, Cin), lambda n, d: (n, 0, 0)),
            pl.BlockSpec((Cin, Cmid), lambda n, d: (0, 0)),
            pl.BlockSpec((4, Cmid), lambda n, d: (0, 0)),
            pl.BlockSpec((HW, 9), lambda n, d: (0, 0)),
            pl.BlockSpec((3, 9 * Cmid, Cmid), lambda n, d: (0, 0, 0)),
            pl.BlockSpec((4, Cmid), lambda n, d: (0, 0)),
        ],
        out_specs=pl.BlockSpec((1, Dt, HW, Cmid), lambda n, d: (n, d, 0, 0)),
        scratch_shapes=[
            pltpu.VMEM((HW, 9 * Cmid), jnp.bfloat16),
            pltpu.VMEM((HW, 3 * Cmid), jnp.float32),
        ],
        compiler_params=pltpu.CompilerParams(
            dimension_semantics=("parallel", "arbitrary")),
    )(x3, w1.astype(jnp.float32), bn2, masks, w3r, bn3)

    return out.reshape(N, D, H, Wd, Cmid).transpose(0, 4, 1, 2, 3)


def _base_kernel(x_ref, w_ref, b_ref, o_ref):
    acc = jnp.dot(x_ref[...], w_ref[...], preferred_element_type=jnp.float32)
    o_ref[...] = jnp.maximum(acc + b_ref[...], 0.0)


def _base_matmul(x, w, b, tn=4096):
    M, K = x.shape
    K2, Nc = w.shape
    assert K == K2
    tn = min(tn, Nc)
    if Nc % tn != 0:
        tn = Nc
    b2 = jnp.reshape(b, (1, Nc)).astype(jnp.float32)
    return pl.pallas_call(
        _base_kernel,
        out_shape=jax.ShapeDtypeStruct((M, Nc), jnp.float32),
        grid=(Nc // tn,),
        in_specs=[
            pl.BlockSpec((M, K), lambda j: (0, 0)),
            pl.BlockSpec((K, tn), lambda j: (0, j)),
            pl.BlockSpec((1, tn), lambda j: (0, j)),
        ],
        out_specs=pl.BlockSpec((M, tn), lambda j: (0, j)),
        compiler_params=pltpu.CompilerParams(
            dimension_semantics=("parallel",)),
    )(x.astype(jnp.float32), w.astype(jnp.float32), b2)


def kernel(l0_w1, l0_bn2d_gamma, l0_bn2d_beta, l0_bn2d_mean, l0_bn2d_var,
           l0_w3, l0_bn3d_gamma, l0_bn3d_beta, l0_bn3d_mean, l0_bn3d_var,
           l1_w1, l1_bn2d_gamma, l1_bn2d_beta, l1_bn2d_mean, l1_bn2d_var,
           l1_w3, l1_bn3d_gamma, l1_bn3d_beta, l1_bn3d_mean, l1_bn3d_var,
           base_w, base_b, feat0, feat1, final_vector):
    out0 = _linker(feat0, l0_w1,
                   (l0_bn2d_gamma, l0_bn2d_beta, l0_bn2d_mean, l0_bn2d_var),
                   l0_w3,
                   (l0_bn3d_gamma, l0_bn3d_beta, l0_bn3d_mean, l0_bn3d_var))
    out1 = _linker(feat1, l1_w1,
                   (l1_bn2d_gamma, l1_bn2d_beta, l1_bn2d_mean, l1_bn2d_var),
                   l1_w3,
                   (l1_bn3d_gamma, l1_bn3d_beta, l1_bn3d_mean, l1_bn3d_var))
    N = final_vector.shape[0]
    flat = final_vector.reshape(N, -1)
    x = _base_matmul(flat, base_w, base_b)
    return x, [out0, out1]
